# Initial kernel scaffold; baseline (speedup 1.0000x reference)
import jax
import jax.numpy as jnp
from jax.experimental import pallas as pl


def kernel(x, edge_index, batch, params):
    raise NotImplementedError("write your pallas kernel here")


# speedup vs baseline: 6.4559x; 6.4559x over previous
import functools

import jax
import jax.numpy as jnp
from jax import lax
from jax.experimental import pallas as pl
from jax.experimental.pallas import tpu as pltpu
from jax.experimental.pallas import tpu_sc as plsc

N = 10000
NP = 10240
E = 320000
H = 128
B = 128
NSUB = 16
NB = 160
CHK = 16
NCH = NB // CHK
EP = NSUB * NB * B
ROWS = NP // NSUB
BLK = 640
GRID = NP // BLK
TBLK = 400
TGRID = N // TBLK
EPS_MSG = 1e-7
EPS_DEN = 1e-16
F32 = jnp.float32


def _dot(a, b):
    return jnp.dot(a, b, preferred_element_type=F32)


def _ln(v, g, b, eps=1e-5):
    mu = jnp.mean(v, axis=-1, keepdims=True)
    var = jnp.mean((v - mu) ** 2, axis=-1, keepdims=True)
    return (v - mu) * jax.lax.rsqrt(var + eps) * g + b


def _table(h, t):
    m = jnp.maximum(h, 0.0) + EPS_MSG
    p = jnp.exp(m * t)
    w = m * p
    return jnp.stack([
        jnp.concatenate([p[:, :64], w[:, :64]], axis=1),
        jnp.concatenate([p[:, 64:], w[:, 64:]], axis=1),
    ])


def _pre_body(x_ref, w_ref, b_ref, t_ref, h_ref, tab_ref):
    h = jnp.maximum(_dot(x_ref[...], w_ref[...]) + b_ref[...], 0.0)
    h_ref[...] = h
    tab_ref[...] = _table(h, t_ref[0, 0])


def _pre_call(xp, fc_w, fc_b, t0):
    return pl.pallas_call(
        _pre_body,
        grid=(GRID,),
        in_specs=[
            pl.BlockSpec((BLK, H), lambda i: (i, 0)),
            pl.BlockSpec((H, H), lambda i: (0, 0)),
            pl.BlockSpec((1, H), lambda i: (0, 0)),
            pl.BlockSpec((1, 1), lambda i: (0, 0)),
        ],
        out_specs=[
            pl.BlockSpec((BLK, H), lambda i: (i, 0)),
            pl.BlockSpec((2, BLK, H), lambda i: (0, i, 0)),
        ],
        out_shape=[
            jax.ShapeDtypeStruct((NP, H), F32),
            jax.ShapeDtypeStruct((2, NP, H), F32),
        ],
        compiler_params=pltpu.CompilerParams(
            dimension_semantics=("arbitrary",)),
    )(xp, fc_w, fc_b, t0)


def _agg_from_sw(sw0, sw1):
    return jnp.concatenate([
        sw0[:, 64:] / (sw0[:, :64] + EPS_DEN),
        sw1[:, 64:] / (sw1[:, :64] + EPS_DEN),
    ], axis=1)


def _mlp_body(sw_ref, h_ref, w1_ref, b1_ref, lng_ref, lnb_ref, w2_ref,
              b2_ref, nrmg_ref, nrmb_ref, t_ref, hn_ref, tab_ref, *, mode):
    agg = _agg_from_sw(sw_ref[0], sw_ref[1])
    y = agg + h_ref[...]
    hm = _dot(y, w1_ref[...]) + b1_ref[...]
    u = jnp.maximum(_ln(hm, lng_ref[...], lnb_ref[...]), 0.0)
    v = _dot(u, w2_ref[...]) + b2_ref[...]
    if mode == 0:
        hn = v
    else:
        hn = h_ref[...] + jnp.maximum(
            _ln(v, nrmg_ref[...], nrmb_ref[...]), 0.0)
    hn_ref[...] = hn
    if tab_ref is not None:
        tab_ref[...] = _table(hn, t_ref[0, 0])


def _mlp_body_notab(sw_ref, h_ref, w1_ref, b1_ref, lng_ref, lnb_ref, w2_ref,
                    b2_ref, nrmg_ref, nrmb_ref, hn_ref, *, mode):
    _mlp_body(sw_ref, h_ref, w1_ref, b1_ref, lng_ref, lnb_ref, w2_ref,
              b2_ref, nrmg_ref, nrmb_ref, None, hn_ref, None, mode=mode)


def _mlp_call(mode, sw, h, c, t_next):
    has_tab = t_next is not None
    in_specs = [
        pl.BlockSpec((2, BLK, H), lambda i: (0, i, 0)),
        pl.BlockSpec((BLK, H), lambda i: (i, 0)),
        pl.BlockSpec((H, 2 * H), lambda i: (0, 0)),
        pl.BlockSpec((1, 2 * H), lambda i: (0, 0)),
        pl.BlockSpec((1, 2 * H), lambda i: (0, 0)),
        pl.BlockSpec((1, 2 * H), lambda i: (0, 0)),
        pl.BlockSpec((2 * H, H), lambda i: (0, 0)),
        pl.BlockSpec((1, H), lambda i: (0, 0)),
        pl.BlockSpec((1, H), lambda i: (0, 0)),
        pl.BlockSpec((1, H), lambda i: (0, 0)),
    ]
    args = [sw, h, c["w1"], c["b1"].reshape(1, -1), c["ln_g"].reshape(1, -1),
            c["ln_b"].reshape(1, -1), c["w2"], c["b2"].reshape(1, -1),
            c["nrm_g"].reshape(1, -1), c["nrm_b"].reshape(1, -1)]
    out_specs = [pl.BlockSpec((BLK, H), lambda i: (i, 0))]
    out_shape = [jax.ShapeDtypeStruct((NP, H), F32)]
    if has_tab:
        in_specs.append(pl.BlockSpec((1, 1), lambda i: (0, 0)))
        args.append(t_next)
        out_specs.append(pl.BlockSpec((2, BLK, H), lambda i: (0, i, 0)))
        out_shape.append(jax.ShapeDtypeStruct((2, NP, H), F32))
        body = functools.partial(_mlp_body, mode=mode)
    else:
        body = functools.partial(_mlp_body_notab, mode=mode)
    res = pl.pallas_call(
        body,
        grid=(GRID,),
        in_specs=in_specs,
        out_specs=out_specs,
        out_shape=out_shape,
        compiler_params=pltpu.CompilerParams(
            dimension_semantics=("arbitrary",)),
    )(*args)
    return res if has_tab else (res[0], None)


def _tail_body(h0_ref, h1_ref, h2_ref, h3_ref, phiw_ref, phib_ref, aw_ref,
               ab_ref, bw_ref, bb_ref, cw_ref, cb_ref, rhow_ref, rhob_ref,
               clsw_ref, clsb_ref, out_ref, acc_ref, stat_ref):
    i = pl.program_id(0)

    @pl.when(i == 0)
    def _init():
        stat_ref[0] = -1e30
        stat_ref[1] = 0.0
        acc_ref[0:1, :] = jnp.zeros((1, 4 * H), F32)

    xc = jnp.concatenate(
        [h0_ref[...], h1_ref[...], h2_ref[...], h3_ref[...]], axis=1)
    hp = jnp.maximum(_dot(xc, phiw_ref[...]) + phib_ref[...], 0.0)
    a = jnp.tanh(_dot(hp, aw_ref[...]) + ab_ref[...])
    bg = jax.nn.sigmoid(_dot(hp, bw_ref[...]) + bb_ref[...])
    av = _dot(a * bg, cw_ref[...]) + cb_ref[...]
    tm = jnp.max(av)
    m_old = stat_ref[0]
    m_new = jnp.maximum(m_old, tm)
    alpha = jnp.exp(m_old - m_new)
    e = jnp.exp(av - m_new)
    stat_ref[0] = m_new
    stat_ref[1] = stat_ref[1] * alpha + jnp.sum(e)
    acc_ref[0:1, :] = acc_ref[0:1, :] * alpha + jnp.sum(
        e * hp, axis=0, keepdims=True)

    @pl.when(i == TGRID - 1)
    def _fin():
        hpool = acc_ref[0:1, :] / stat_ref[1]
        hres = jnp.maximum(_dot(hpool, rhow_ref[...]) + rhob_ref[...], 0.0)
        logit = _dot(hres, clsw_ref[...]) + clsb_ref[...]
        out_ref[...] = jax.nn.sigmoid(logit)


def _tail_call(h0, h1, h2, h3, p):
    D4 = 4 * H
    full = lambda r, c: pl.BlockSpec((r, c), lambda i: (0, 0))
    blk = pl.BlockSpec((TBLK, H), lambda i: (i, 0))
    return pl.pallas_call(
        _tail_body,
        grid=(TGRID,),
        in_specs=[blk, blk, blk, blk,
                  full(D4, D4), full(1, D4),
                  full(D4, D4), full(1, D4),
                  full(D4, D4), full(1, D4),
                  full(D4, 1), full(1, 1),
                  full(D4, D4), full(1, D4),
                  full(D4, 1), full(1, 1)],
        out_specs=pl.BlockSpec((1, 1), lambda i: (0, 0)),
        out_shape=jax.ShapeDtypeStruct((1, 1), F32),
        scratch_shapes=[
            pltpu.VMEM((8, D4), F32),
            pltpu.SMEM((2,), F32),
        ],
        compiler_params=pltpu.CompilerParams(
            dimension_semantics=("arbitrary",)),
    )(h0, h1, h2, h3,
      p["phi_w"], p["phi_b"].reshape(1, -1),
      p["aw"], p["ab"].reshape(1, -1),
      p["bw"], p["bb"].reshape(1, -1),
      p["cw"], p["cb"].reshape(1, -1),
      p["rho_w"], p["rho_b"].reshape(1, -1),
      p["cls_w"], p["cls_b"].reshape(1, -1))


@functools.cache
def _make_agg_kernel():
    mesh = plsc.VectorSubcoreMesh(core_axis_name="c", subcore_axis_name="s")

    @functools.partial(
        pl.kernel,
        mesh=mesh,
        out_type=jax.ShapeDtypeStruct((2, NP, H), F32),
        scratch_types=[
            pltpu.VMEM((CHK, B), jnp.int32),
            pltpu.VMEM((CHK, B), jnp.int32),
            pltpu.VMEM((B, H), F32),
            pltpu.VMEM_SHARED((NP, H), F32),
            pltpu.SemaphoreType.DMA,
        ],
    )
    def agg(tab, srcr, dstr, zeros, out, sidx, didx, buf, acc, sem):
        c = lax.axis_index("c")
        s = lax.axis_index("s")
        row0 = s * ROWS
        pltpu.sync_copy(zeros.at[pl.ds(row0, ROWS)],
                        acc.at[pl.ds(row0, ROWS)])
        plsc.subcore_barrier()

        def chunk(k, carry):
            pltpu.sync_copy(srcr.at[c, s, k], sidx)
            pltpu.sync_copy(dstr.at[s, k], didx)

            def body(j, carry2):
                pltpu.async_copy(tab.at[sidx.at[j]], buf, sem).wait()
                pltpu.sync_copy(buf, acc.at[didx.at[j]], add=True)
                return carry2

            return lax.fori_loop(0, CHK, body, carry)

        lax.fori_loop(0, NCH, chunk, 0)
        plsc.subcore_barrier()
        pltpu.sync_copy(acc.at[pl.ds(row0, ROWS)],
                        out.at[c, pl.ds(row0, ROWS)])

    return agg


def _agg_call(tab_flat, srcr, dstr, zeros):
    return _make_agg_kernel()(tab_flat, srcr, dstr, zeros)


def kernel(x, edge_index, batch, params):
    del batch
    xp = jnp.pad(x, ((0, NP - N), (0, 0)))
    src = edge_index[0]
    dst = edge_index[1]
    src_p = jnp.concatenate([src, jnp.zeros((EP - E,), jnp.int32)])
    dst_p = jnp.concatenate([dst, jnp.full((EP - E,), N, jnp.int32)])
    srcr = jnp.stack([src_p, src_p + NP]).reshape(2, NSUB, NCH, CHK, B)
    dstr = dst_p.reshape(NSUB, NCH, CHK, B)
    zeros = jnp.zeros((NP, H), F32)
    cs = params["convs"]
    ts = [c["t"].reshape(1, 1) for c in cs]

    h_fc, tab = _pre_call(xp, params["fc_w"], params["fc_b"].reshape(1, -1),
                          ts[0])
    sw = _agg_call(tab.reshape(2 * NP, H), srcr, dstr, zeros)
    h1, tab = _mlp_call(0, sw, h_fc, cs[0], ts[1])
    sw = _agg_call(tab.reshape(2 * NP, H), srcr, dstr, zeros)
    h2, tab = _mlp_call(1, sw, h1, cs[1], ts[2])
    sw = _agg_call(tab.reshape(2 * NP, H), srcr, dstr, zeros)
    h3, _ = _mlp_call(1, sw, h2, cs[2], None)
    return _tail_call(h_fc, h1, h2, h3, params)
